# Initial kernel scaffold; baseline (speedup 1.0000x reference)
import jax
import jax.numpy as jnp
from jax.experimental import pallas as pl


def kernel(idx, codebook_B, codebook_H, W, b, gamma, beta):
    raise NotImplementedError("write your pallas kernel here")


# speedup vs baseline: 3.3869x; 3.3869x over previous
import functools

import jax
import jax.numpy as jnp
from jax import lax
from jax.experimental import pallas as pl
from jax.experimental.pallas import tpu as pltpu
from jax.experimental.pallas import tpu_sc as plsc

_K = 8192
_CODE_DIM = 32
_TOKENS = 8
_BATCH = 1024
_T = 50
_D_MODEL = 768
_FAN_IN = _TOKENS * _CODE_DIM

_ROWS = _BATCH * _T * _TOKENS
_NW = 32
_ROWS_PER_W = _ROWS // _NW
_STREAM = 128
_STREAMS_PER_BLK = 20
_BLK_ROWS = _STREAM * _STREAMS_PER_BLK
_BLKS = _ROWS_PER_W // _BLK_ROWS


def _sc_gather(table, idx3d):
    mesh = plsc.VectorSubcoreMesh(core_axis_name="c", subcore_axis_name="s")
    streams_per_w = _ROWS_PER_W // _STREAM

    @functools.partial(
        pl.kernel,
        mesh=mesh,
        compiler_params=pltpu.CompilerParams(use_tc_tiling_on_sc=False),
        out_type=jax.ShapeDtypeStruct((_ROWS, _CODE_DIM), jnp.float32),
        scratch_types=[
            pltpu.VMEM((streams_per_w, _STREAM), jnp.int32),
            pltpu.VMEM((_BLK_ROWS, _CODE_DIM), jnp.float32),
            pltpu.VMEM_SHARED((2 * _K, _CODE_DIM), jnp.float32),
            pltpu.SemaphoreType.DMA,
        ],
    )
    def k(table_hbm, idx_hbm, out_hbm, idx_v, rows_v, table_sp, sem):
        cid = lax.axis_index("c")
        sid = lax.axis_index("s")
        wid = sid * 2 + cid
        row_base = wid * _ROWS_PER_W

        stage = (2 * _K) // 16
        pltpu.sync_copy(
            table_hbm.at[pl.ds(sid * stage, stage)],
            table_sp.at[pl.ds(sid * stage, stage)],
        )
        pltpu.sync_copy(idx_hbm.at[wid], idx_v)
        plsc.subcore_barrier()

        def body(blk, carry):
            copies = []
            for i in range(_STREAMS_PER_BLK):
                copies.append(
                    pltpu.async_copy(
                        table_sp.at[idx_v.at[blk * _STREAMS_PER_BLK + i]],
                        rows_v.at[pl.ds(i * _STREAM, _STREAM)],
                        sem,
                    )
                )
            for c in copies:
                c.wait()
            pltpu.sync_copy(
                rows_v,
                out_hbm.at[pl.ds(row_base + blk * _BLK_ROWS, _BLK_ROWS)],
            )
            return carry

        lax.fori_loop(0, _BLKS, body, 0)

    return k(table, idx3d)


_BB = 8
_BLK = _BB * _T


def _tc_body(z_ref, w_ref, b_ref, g_ref, bt_ref, out_ref, pool_ref):
    y = jnp.dot(z_ref[...], w_ref[...], preferred_element_type=jnp.float32)
    y = y + b_ref[...]
    mean = jnp.mean(y, axis=-1, keepdims=True)
    var = jnp.mean((y - mean) ** 2, axis=-1, keepdims=True)
    zn = (y - mean) * lax.rsqrt(var + 1e-5) * g_ref[...] + bt_ref[...]
    out_ref[...] = zn
    pool_ref[...] = jnp.mean(zn.reshape(_BB, _T, _D_MODEL), axis=1)


def _tc_fuse(zflat, W, b, gamma, beta):
    grid = (_BATCH // _BB,)
    return pl.pallas_call(
        _tc_body,
        grid=grid,
        in_specs=[
            pl.BlockSpec((_BLK, _FAN_IN), lambda i: (i, 0)),
            pl.BlockSpec((_FAN_IN, _D_MODEL), lambda i: (0, 0)),
            pl.BlockSpec((1, _D_MODEL), lambda i: (0, 0)),
            pl.BlockSpec((1, _D_MODEL), lambda i: (0, 0)),
            pl.BlockSpec((1, _D_MODEL), lambda i: (0, 0)),
        ],
        out_specs=[
            pl.BlockSpec((_BLK, _D_MODEL), lambda i: (i, 0)),
            pl.BlockSpec((_BB, _D_MODEL), lambda i: (i, 0)),
        ],
        out_shape=[
            jax.ShapeDtypeStruct((_BATCH * _T, _D_MODEL), jnp.float32),
            jax.ShapeDtypeStruct((_BATCH, _D_MODEL), jnp.float32),
        ],
    )(zflat, W, b.reshape(1, -1), gamma.reshape(1, -1), beta.reshape(1, -1))


def kernel(idx, codebook_B, codebook_H, W, b, gamma, beta):
    table = jnp.concatenate([codebook_B, codebook_H], axis=0)
    offs = jnp.where(jnp.arange(_TOKENS, dtype=jnp.int32) >= 4, _K, 0)
    idx_adj = (idx.reshape(_BATCH, _T, _TOKENS) + offs[None, None, :])
    idx3d = idx_adj.reshape(_NW, _ROWS_PER_W // _STREAM, _STREAM)

    emb = _sc_gather(table, idx3d)
    zflat = emb.reshape(_BATCH * _T, _FAN_IN)
    z2d, pooled = _tc_fuse(zflat, W, b, gamma, beta)
    z = z2d.reshape(_BATCH, _T, _D_MODEL)
    return (z, pooled)
